# baseline (speedup 1.0000x reference)
import functools
import math

import jax
import jax.numpy as jnp
from jax import lax
from jax.experimental import pallas as pl
from jax.experimental.pallas import tpu as pltpu
from jax.experimental.pallas import tpu_sc as plsc

_N = 10000
_NPAD = 10112
_E = 160000
_D = 128
_ACT_NORM = 0.5595
_AVG = 32.0

_NC, _NS = 2, 16
_NW = _NC * _NS

_GW = 40
_G_WPW = _E // _GW // _NW
_SW = 40
_S_WPT = _E // _SW // _NS
_N_PT = _NPAD // _NS
_NBUF = 5
_SNBUF = 5
_S_NG = _S_WPT // _SNBUF

_BE = 3200
_E1 = 76800
_E2 = _E - _E1


def _gather_body(node_hbm, send_hbm, sg_hbm, sidx_v, gbuf, nodes_s, sem_in, sem_out):
    cid = lax.axis_index("c")
    sid = lax.axis_index("s")
    wid = sid * _NC + cid

    @pl.when(sid < _NS - 1)
    def _():
        pltpu.sync_copy(
            node_hbm.at[pl.ds(sid * 632, 632)], nodes_s.at[pl.ds(sid * 632, 632)]
        )

    @pl.when(sid == _NS - 1)
    def _():
        pltpu.sync_copy(
            node_hbm.at[pl.ds(9480, _N - 9480)], nodes_s.at[pl.ds(9480, _N - 9480)]
        )

    pltpu.sync_copy(send_hbm.at[wid], sidx_v)
    plsc.subcore_barrier()

    @pl.loop(0, _G_WPW, step=_NBUF)
    def grp(w0):
        for b in range(_NBUF):
            @pl.when(w0 > 0)
            def _():
                pltpu.make_async_copy(
                    gbuf.at[b],
                    sg_hbm.at[pl.ds((wid * _G_WPW) * _GW, _GW)],
                    sem_out.at[b],
                ).wait()

            pltpu.async_copy(nodes_s.at[sidx_v.at[w0 + b]], gbuf.at[b], sem_in.at[b])
        for b in range(_NBUF):
            pltpu.make_async_copy(
                nodes_s.at[sidx_v.at[w0 + b]], gbuf.at[b], sem_in.at[b]
            ).wait()
            pltpu.async_copy(
                gbuf.at[b],
                sg_hbm.at[pl.ds((wid * _G_WPW + w0 + b) * _GW, _GW)],
                sem_out.at[b],
            )

    for b in range(_NBUF):
        pltpu.make_async_copy(
            gbuf.at[b],
            sg_hbm.at[pl.ds((wid * _G_WPW) * _GW, _GW)],
            sem_out.at[b],
        ).wait()


def _sc_gather(node_feats, send3d):
    mesh = plsc.VectorSubcoreMesh(core_axis_name="c", subcore_axis_name="s")
    f = functools.partial(
        pl.kernel,
        out_type=jax.ShapeDtypeStruct((_E, _D), jnp.float32),
        mesh=mesh,
        scratch_types=[
            pltpu.VMEM((_G_WPW, _GW), jnp.int32),
            pltpu.VMEM((_NBUF, _GW, _D), jnp.float32),
            pltpu.VMEM_SHARED((_N, _D), jnp.float32),
            pltpu.SemaphoreType.DMA((_NBUF,)),
            pltpu.SemaphoreType.DMA((_NBUF,)),
        ],
    )(_gather_body)
    return f(node_feats, send3d)


def _tc_body(rad_ref, vec_ref, sg_ref, w0, w1, w2, w3, out_ref):
    rad = rad_ref[...]
    h = jnp.dot(rad, w0[...], preferred_element_type=jnp.float32)
    h = jax.nn.silu(h) * (1.0 / _ACT_NORM)
    h = jnp.dot(h, w1[...], preferred_element_type=jnp.float32)
    h = jax.nn.silu(h) * (1.0 / _ACT_NORM)
    h = jnp.dot(h, w2[...], preferred_element_type=jnp.float32)
    h = jax.nn.silu(h) * (1.0 / _ACT_NORM)
    mix = jnp.dot(h, w3[...], preferred_element_type=jnp.float32)

    v = -vec_ref[...]
    nrm = jnp.sqrt(jnp.sum(v * v, axis=-1, keepdims=True))
    sh = (math.sqrt(3.0) / _AVG) * v / (nrm + 1e-9)

    g = sg_ref[...]
    a = g * mix[:, :_D] * (1.0 / _AVG)
    b = g * mix[:, _D:]
    out_ref[0] = a
    out_ref[1] = b * sh[:, 0:1]
    out_ref[2] = b * sh[:, 1:2]
    out_ref[3] = b * sh[:, 2:3]


def _tc_messages(radial, vectors, sg, W0s, W1s, W2s, W3s, e_half, blk_off):
    nblk = e_half // _BE
    return pl.pallas_call(
        _tc_body,
        grid=(nblk,),
        in_specs=[
            pl.BlockSpec((_BE, 8), lambda i, o=blk_off: (i + o, 0)),
            pl.BlockSpec((_BE, 3), lambda i, o=blk_off: (i + o, 0)),
            pl.BlockSpec((_BE, _D), lambda i, o=blk_off: (i + o, 0)),
            pl.BlockSpec((8, 64), lambda i: (0, 0)),
            pl.BlockSpec((64, 64), lambda i: (0, 0)),
            pl.BlockSpec((64, 64), lambda i: (0, 0)),
            pl.BlockSpec((64, 2 * _D), lambda i: (0, 0)),
        ],
        out_specs=pl.BlockSpec((4, _BE, _D), lambda i: (0, i, 0)),
        out_shape=jax.ShapeDtypeStruct((4, e_half, _D), jnp.float32),
    )(radial, vectors, sg, W0s, W1s, W2s, W3s)


def _scatter_body(ng, msgs_hbm, recv_hbm, out_hbm, rbufs, bufs, acc, sem_in, sem_out, rsem):
    wpt = ng * _SNBUF
    cid = lax.axis_index("c")
    sid = lax.axis_index("s")

    for cc in range(2):
        chunk = cid * 2 + cc

        def zb(i, carry):
            bufs[0, i // (_D // 16), pl.ds((i % (_D // 16)) * 16, 16)] = jnp.zeros(
                (16,), jnp.float32
            )
            return carry

        lax.fori_loop(0, _SW * (_D // 16), zb, 0)

        def zc(j, carry):
            pltpu.sync_copy(bufs.at[0], acc.at[pl.ds(sid * _N_PT + j * _SW, _SW)])
            return carry

        lax.fori_loop(0, _N_PT // _SW, zc, 0)
        pltpu.sync_copy(
            bufs.at[0, pl.ds(0, _N_PT % _SW)],
            acc.at[pl.ds(sid * _N_PT + (_N_PT // _SW) * _SW, _N_PT % _SW)],
        )
        plsc.subcore_barrier()

        pltpu.sync_copy(recv_hbm.at[sid, 0], rbufs.at[0])

        @pl.loop(0, ng, step=2)
        def grp2(g0):
            for gg in range(2):
                g = g0 + gg

                @pl.when(g > 0)
                def _():
                    pltpu.make_async_copy(
                        recv_hbm.at[sid, 0], rbufs.at[gg], rsem.at[gg]
                    ).wait()

                for b in range(_SNBUF):
                    @pl.when(g > 0)
                    def _():
                        pltpu.make_async_copy(
                            bufs.at[b], acc.at[rbufs.at[0, 0]], sem_out.at[b]
                        ).wait()

                    e0 = sid * (wpt * _SW) + (g * _SNBUF + b) * _SW
                    pltpu.async_copy(
                        msgs_hbm.at[chunk, pl.ds(e0, _SW)], bufs.at[b], sem_in.at[b]
                    )

                @pl.when(g + 1 < ng)
                def _():
                    pltpu.async_copy(
                        recv_hbm.at[sid, g + 1], rbufs.at[1 - gg], rsem.at[1 - gg]
                    )

                for b in range(_SNBUF):
                    e0 = sid * (wpt * _SW) + (g * _SNBUF + b) * _SW
                    pltpu.make_async_copy(
                        msgs_hbm.at[chunk, pl.ds(e0, _SW)], bufs.at[b], sem_in.at[b]
                    ).wait()
                    pltpu.async_copy(
                        bufs.at[b], acc.at[rbufs.at[gg, b]], sem_out.at[b], add=True
                    )

        for b in range(_SNBUF):
            pltpu.make_async_copy(
                bufs.at[b], acc.at[rbufs.at[0, 0]], sem_out.at[b]
            ).wait()

        plsc.subcore_barrier()
        pltpu.sync_copy(
            acc.at[pl.ds(sid * _N_PT, _N_PT)],
            out_hbm.at[chunk, pl.ds(sid * _N_PT, _N_PT)],
        )
        plsc.subcore_barrier()


def _sc_scatter(msgs, recv4d, ng):
    mesh = plsc.VectorSubcoreMesh(core_axis_name="c", subcore_axis_name="s")
    f = functools.partial(
        pl.kernel,
        out_type=jax.ShapeDtypeStruct((4, _NPAD, _D), jnp.float32),
        mesh=mesh,
        scratch_types=[
            pltpu.VMEM((2, _SNBUF, _SW), jnp.int32),
            pltpu.VMEM((_SNBUF, _SW, _D), jnp.float32),
            pltpu.VMEM_SHARED((_NPAD, _D), jnp.float32),
            pltpu.SemaphoreType.DMA((_SNBUF,)),
            pltpu.SemaphoreType.DMA((_SNBUF,)),
            pltpu.SemaphoreType.DMA((2,)),
        ],
    )(functools.partial(_scatter_body, ng))
    return f(msgs, recv4d)


def kernel(vectors, node_feats, radial_embedding, senders, receivers, W0, W1, W2, W3):
    send3d = senders.reshape(_NW, _G_WPW, _GW)
    ng1 = _E1 // _SW // _NS // _SNBUF
    ng2 = _E2 // _SW // _NS // _SNBUF
    recv4d_1 = receivers[:_E1].reshape(_NS, ng1, _SNBUF, _SW)
    recv4d_2 = receivers[_E1:].reshape(_NS, ng2, _SNBUF, _SW)
    W0s = W0 * (1.0 / math.sqrt(W0.shape[0]))
    W1s = W1 * (1.0 / math.sqrt(W1.shape[0]))
    W2s = W2 * (1.0 / math.sqrt(W2.shape[0]))
    W3s = W3 * (1.0 / math.sqrt(W3.shape[0]))

    sg = _sc_gather(node_feats, send3d)
    msgs1 = _tc_messages(radial_embedding, vectors, sg, W0s, W1s, W2s, W3s, _E1, 0)
    out4a = _sc_scatter(msgs1, recv4d_1, ng1)
    msgs2 = _tc_messages(radial_embedding, vectors, sg, W0s, W1s, W2s, W3s, _E2, _E1 // _BE)
    out4b = _sc_scatter(msgs2, recv4d_2, ng2)
    out4 = (out4a + out4b)[:, :_N]

    out_s = out4[0]
    out_v = jnp.transpose(out4[1:], (1, 2, 0)).reshape(_N, 3 * _D)
    return jnp.concatenate([out_s, out_v], axis=-1)


# speedup vs baseline: 1.0050x; 1.0050x over previous
import functools
import math

import jax
import jax.numpy as jnp
from jax import lax
from jax.experimental import pallas as pl
from jax.experimental.pallas import tpu as pltpu
from jax.experimental.pallas import tpu_sc as plsc

_N = 10000
_NPAD = 10112
_E = 160000
_D = 128
_ACT_NORM = 0.5595
_AVG = 32.0

_NC, _NS = 2, 16
_NW = _NC * _NS

_GW = 40
_G_WPW = _E // _GW // _NW
_SW = 40
_S_WPT = _E // _SW // _NS
_N_PT = _NPAD // _NS
_NBUF = 5
_SNBUF = 5
_S_NG = _S_WPT // _SNBUF

_BE = 3200
_E1 = 64000
_E2 = _E - _E1


def _gather_body(node_hbm, send_hbm, sg_hbm, sidx_v, gbuf, nodes_s, sem_in, sem_out):
    cid = lax.axis_index("c")
    sid = lax.axis_index("s")
    wid = sid * _NC + cid

    @pl.when(sid < _NS - 1)
    def _():
        pltpu.sync_copy(
            node_hbm.at[pl.ds(sid * 632, 632)], nodes_s.at[pl.ds(sid * 632, 632)]
        )

    @pl.when(sid == _NS - 1)
    def _():
        pltpu.sync_copy(
            node_hbm.at[pl.ds(9480, _N - 9480)], nodes_s.at[pl.ds(9480, _N - 9480)]
        )

    pltpu.sync_copy(send_hbm.at[wid], sidx_v)
    plsc.subcore_barrier()

    @pl.loop(0, _G_WPW, step=_NBUF)
    def grp(w0):
        for b in range(_NBUF):
            @pl.when(w0 > 0)
            def _():
                pltpu.make_async_copy(
                    gbuf.at[b],
                    sg_hbm.at[pl.ds((wid * _G_WPW) * _GW, _GW)],
                    sem_out.at[b],
                ).wait()

            pltpu.async_copy(nodes_s.at[sidx_v.at[w0 + b]], gbuf.at[b], sem_in.at[b])
        for b in range(_NBUF):
            pltpu.make_async_copy(
                nodes_s.at[sidx_v.at[w0 + b]], gbuf.at[b], sem_in.at[b]
            ).wait()
            pltpu.async_copy(
                gbuf.at[b],
                sg_hbm.at[pl.ds((wid * _G_WPW + w0 + b) * _GW, _GW)],
                sem_out.at[b],
            )

    for b in range(_NBUF):
        pltpu.make_async_copy(
            gbuf.at[b],
            sg_hbm.at[pl.ds((wid * _G_WPW) * _GW, _GW)],
            sem_out.at[b],
        ).wait()


def _sc_gather(node_feats, send3d):
    mesh = plsc.VectorSubcoreMesh(core_axis_name="c", subcore_axis_name="s")
    f = functools.partial(
        pl.kernel,
        out_type=jax.ShapeDtypeStruct((_E, _D), jnp.float32),
        mesh=mesh,
        scratch_types=[
            pltpu.VMEM((_G_WPW, _GW), jnp.int32),
            pltpu.VMEM((_NBUF, _GW, _D), jnp.float32),
            pltpu.VMEM_SHARED((_N, _D), jnp.float32),
            pltpu.SemaphoreType.DMA((_NBUF,)),
            pltpu.SemaphoreType.DMA((_NBUF,)),
        ],
    )(_gather_body)
    return f(node_feats, send3d)


def _tc_body(rad_ref, vec_ref, sg_ref, w0, w1, w2, w3, out_ref):
    rad = rad_ref[...]
    h = jnp.dot(rad, w0[...], preferred_element_type=jnp.float32)
    h = jax.nn.silu(h) * (1.0 / _ACT_NORM)
    h = jnp.dot(h, w1[...], preferred_element_type=jnp.float32)
    h = jax.nn.silu(h) * (1.0 / _ACT_NORM)
    h = jnp.dot(h, w2[...], preferred_element_type=jnp.float32)
    h = jax.nn.silu(h) * (1.0 / _ACT_NORM)
    mix = jnp.dot(h, w3[...], preferred_element_type=jnp.float32)

    v = -vec_ref[...]
    nrm = jnp.sqrt(jnp.sum(v * v, axis=-1, keepdims=True))
    sh = (math.sqrt(3.0) / _AVG) * v / (nrm + 1e-9)

    g = sg_ref[...]
    a = g * mix[:, :_D] * (1.0 / _AVG)
    b = g * mix[:, _D:]
    out_ref[0] = a
    out_ref[1] = b * sh[:, 0:1]
    out_ref[2] = b * sh[:, 1:2]
    out_ref[3] = b * sh[:, 2:3]


def _tc_messages(radial, vectors, sg, W0s, W1s, W2s, W3s, e_half, blk_off):
    nblk = e_half // _BE
    return pl.pallas_call(
        _tc_body,
        grid=(nblk,),
        in_specs=[
            pl.BlockSpec((_BE, 8), lambda i, o=blk_off: (i + o, 0)),
            pl.BlockSpec((_BE, 3), lambda i, o=blk_off: (i + o, 0)),
            pl.BlockSpec((_BE, _D), lambda i, o=blk_off: (i + o, 0)),
            pl.BlockSpec((8, 64), lambda i: (0, 0)),
            pl.BlockSpec((64, 64), lambda i: (0, 0)),
            pl.BlockSpec((64, 64), lambda i: (0, 0)),
            pl.BlockSpec((64, 2 * _D), lambda i: (0, 0)),
        ],
        out_specs=pl.BlockSpec((4, _BE, _D), lambda i: (0, i, 0)),
        out_shape=jax.ShapeDtypeStruct((4, e_half, _D), jnp.float32),
    )(radial, vectors, sg, W0s, W1s, W2s, W3s)


def _scatter_body(ng, msgs_hbm, recv_hbm, init_hbm, out_hbm, rbufs, bufs, acc, sem_in, sem_out, rsem):
    wpt = ng * _SNBUF
    cid = lax.axis_index("c")
    sid = lax.axis_index("s")

    for cc in range(2):
        chunk = cid * 2 + cc

        pltpu.sync_copy(
            init_hbm.at[chunk, pl.ds(sid * _N_PT, _N_PT)],
            acc.at[pl.ds(sid * _N_PT, _N_PT)],
        )
        plsc.subcore_barrier()

        pltpu.sync_copy(recv_hbm.at[sid, 0], rbufs.at[0])

        @pl.loop(0, ng, step=2)
        def grp2(g0):
            for gg in range(2):
                g = g0 + gg

                @pl.when(g > 0)
                def _():
                    pltpu.make_async_copy(
                        recv_hbm.at[sid, 0], rbufs.at[gg], rsem.at[gg]
                    ).wait()

                for b in range(_SNBUF):
                    @pl.when(g > 0)
                    def _():
                        pltpu.make_async_copy(
                            bufs.at[b], acc.at[rbufs.at[0, 0]], sem_out.at[b]
                        ).wait()

                    e0 = sid * (wpt * _SW) + (g * _SNBUF + b) * _SW
                    pltpu.async_copy(
                        msgs_hbm.at[chunk, pl.ds(e0, _SW)], bufs.at[b], sem_in.at[b]
                    )

                @pl.when(g + 1 < ng)
                def _():
                    pltpu.async_copy(
                        recv_hbm.at[sid, g + 1], rbufs.at[1 - gg], rsem.at[1 - gg]
                    )

                for b in range(_SNBUF):
                    e0 = sid * (wpt * _SW) + (g * _SNBUF + b) * _SW
                    pltpu.make_async_copy(
                        msgs_hbm.at[chunk, pl.ds(e0, _SW)], bufs.at[b], sem_in.at[b]
                    ).wait()
                    pltpu.async_copy(
                        bufs.at[b], acc.at[rbufs.at[gg, b]], sem_out.at[b], add=True
                    )

        for b in range(_SNBUF):
            pltpu.make_async_copy(
                bufs.at[b], acc.at[rbufs.at[0, 0]], sem_out.at[b]
            ).wait()

        plsc.subcore_barrier()
        pltpu.sync_copy(
            acc.at[pl.ds(sid * _N_PT, _N_PT)],
            out_hbm.at[chunk, pl.ds(sid * _N_PT, _N_PT)],
        )
        plsc.subcore_barrier()


def _sc_scatter(msgs, recv4d, ng, init):
    mesh = plsc.VectorSubcoreMesh(core_axis_name="c", subcore_axis_name="s")
    f = functools.partial(
        pl.kernel,
        out_type=jax.ShapeDtypeStruct((4, _NPAD, _D), jnp.float32),
        mesh=mesh,
        scratch_types=[
            pltpu.VMEM((2, _SNBUF, _SW), jnp.int32),
            pltpu.VMEM((_SNBUF, _SW, _D), jnp.float32),
            pltpu.VMEM_SHARED((_NPAD, _D), jnp.float32),
            pltpu.SemaphoreType.DMA((_SNBUF,)),
            pltpu.SemaphoreType.DMA((_SNBUF,)),
            pltpu.SemaphoreType.DMA((2,)),
        ],
    )(functools.partial(_scatter_body, ng))
    return f(msgs, recv4d, init)


def kernel(vectors, node_feats, radial_embedding, senders, receivers, W0, W1, W2, W3):
    send3d = senders.reshape(_NW, _G_WPW, _GW)
    ng1 = _E1 // _SW // _NS // _SNBUF
    ng2 = _E2 // _SW // _NS // _SNBUF
    recv4d_1 = receivers[:_E1].reshape(_NS, ng1, _SNBUF, _SW)
    recv4d_2 = receivers[_E1:].reshape(_NS, ng2, _SNBUF, _SW)
    W0s = W0 * (1.0 / math.sqrt(W0.shape[0]))
    W1s = W1 * (1.0 / math.sqrt(W1.shape[0]))
    W2s = W2 * (1.0 / math.sqrt(W2.shape[0]))
    W3s = W3 * (1.0 / math.sqrt(W3.shape[0]))

    sg = _sc_gather(node_feats, send3d)
    zeros4 = jnp.zeros((4, _NPAD, _D), jnp.float32)
    msgs1 = _tc_messages(radial_embedding, vectors, sg, W0s, W1s, W2s, W3s, _E1, 0)
    out4a = _sc_scatter(msgs1, recv4d_1, ng1, zeros4)
    msgs2 = _tc_messages(radial_embedding, vectors, sg, W0s, W1s, W2s, W3s, _E2, _E1 // _BE)
    out4 = _sc_scatter(msgs2, recv4d_2, ng2, out4a)[:, :_N]

    out_s = out4[0]
    out_v = jnp.transpose(out4[1:], (1, 2, 0)).reshape(_N, 3 * _D)
    return jnp.concatenate([out_s, out_v], axis=-1)
